# Initial kernel scaffold; baseline (speedup 1.0000x reference)
import functools

import jax
import jax.numpy as jnp
from jax import lax
from jax.experimental import pallas as pl
from jax.experimental.pallas import tpu as pltpu
from jax.experimental.pallas import tpu_sc as plsc

NC = 2
NS = 16
CHUNK = 128


def _sc_degree(dst3, zeros16, ones16, n_acc, rpt, ch_deg):
    mesh = plsc.VectorSubcoreMesh(core_axis_name="c", subcore_axis_name="s")

    @functools.partial(
        pl.kernel,
        out_type=jax.ShapeDtypeStruct((NC, n_acc, 16), jnp.float32),
        mesh=mesh,
        scratch_types=[
            pltpu.VMEM((ch_deg, CHUNK), jnp.int32),
            pltpu.VMEM((CHUNK, 16), jnp.float32),
            pltpu.VMEM_SHARED((n_acc, 16), jnp.float32),
        ],
    )
    def deg_kernel(dst_hbm, zero_hbm, one_hbm, degp_hbm, dstv, onesv, acc):
        c = lax.axis_index("c")
        s = lax.axis_index("s")
        t = c * NS + s
        r0 = s * rpt
        pltpu.sync_copy(zero_hbm.at[pl.ds(r0, rpt)], acc.at[pl.ds(r0, rpt)])
        pltpu.sync_copy(one_hbm, onesv)
        pltpu.sync_copy(dst_hbm.at[t], dstv)
        plsc.subcore_barrier()

        def body(j, carry):
            pltpu.sync_copy(onesv, acc.at[dstv.at[j]], add=True)
            return carry

        lax.fori_loop(0, ch_deg, body, 0)
        plsc.subcore_barrier()
        pltpu.sync_copy(acc.at[pl.ds(r0, rpt)], degp_hbm.at[c, pl.ds(r0, rpt)])

    return deg_kernel(dst3, zeros16, ones16)


def _sc_scatter(g0, g1, src3, dst3, zeros_half, n_acc, rpt, ch_mp, half):
    mesh = plsc.VectorSubcoreMesh(core_axis_name="c", subcore_axis_name="s")

    @functools.partial(
        pl.kernel,
        out_type=jax.ShapeDtypeStruct((NC, n_acc, half), jnp.float32),
        mesh=mesh,
        scratch_types=[
            pltpu.VMEM((ch_mp, CHUNK), jnp.int32),
            pltpu.VMEM((ch_mp, CHUNK), jnp.int32),
            pltpu.VMEM((2, CHUNK, half), jnp.float32),
            pltpu.VMEM_SHARED((n_acc, half), jnp.float32),
            pltpu.SemaphoreType.DMA,
            pltpu.SemaphoreType.DMA,
        ],
    )
    def mp_kernel(g0_hbm, g1_hbm, src_hbm, dst_hbm, zero_hbm, agg_hbm,
                  srcv, dstv, rowsv, acc, sem0, sem1):
        c = lax.axis_index("c")
        s = lax.axis_index("s")
        r0 = s * rpt
        pltpu.sync_copy(zero_hbm.at[pl.ds(r0, rpt)], acc.at[pl.ds(r0, rpt)])
        pltpu.sync_copy(src_hbm.at[s], srcv)
        pltpu.sync_copy(dst_hbm.at[s], dstv)
        plsc.subcore_barrier()

        def run(g_tab):
            pltpu.async_copy(g_tab.at[srcv.at[0]], rowsv.at[0], sem0)
            pltpu.async_copy(g_tab.at[srcv.at[1]], rowsv.at[1], sem1)

            def body(jj, carry):
                j0 = 2 * jj
                pltpu.make_async_copy(g_tab.at[srcv.at[j0]], rowsv.at[0],
                                      sem0).wait()
                pltpu.sync_copy(rowsv.at[0], acc.at[dstv.at[j0]], add=True)

                @pl.when(j0 + 2 < ch_mp)
                def _():
                    pltpu.async_copy(g_tab.at[srcv.at[j0 + 2]], rowsv.at[0],
                                     sem0)

                pltpu.make_async_copy(g_tab.at[srcv.at[j0 + 1]], rowsv.at[1],
                                      sem1).wait()
                pltpu.sync_copy(rowsv.at[1], acc.at[dstv.at[j0 + 1]], add=True)

                @pl.when(j0 + 3 < ch_mp)
                def _():
                    pltpu.async_copy(g_tab.at[srcv.at[j0 + 3]], rowsv.at[1],
                                     sem1)

                return carry

            lax.fori_loop(0, ch_mp // 2, body, 0)

        @pl.when(c == 0)
        def _():
            run(g0_hbm)

        @pl.when(c == 1)
        def _():
            run(g1_hbm)

        plsc.subcore_barrier()
        pltpu.sync_copy(acc.at[pl.ds(r0, rpt)], agg_hbm.at[c, pl.ds(r0, rpt)])

    return mp_kernel(g0, g1, src3, dst3, zeros_half)


def _tc_matmul(x, W, degp, bm):
    n, d_in = x.shape
    d_out = W.shape[1]

    def mm_body(x_ref, w_ref, d_ref, g_ref):
        deg = 1.0 + d_ref[0, :, 0] + d_ref[1, :, 0]
        dinv = lax.rsqrt(deg)
        h = jnp.dot(x_ref[...], w_ref[...], preferred_element_type=jnp.float32)
        g_ref[...] = h * dinv[:, None]

    return pl.pallas_call(
        mm_body,
        grid=(n // bm,),
        in_specs=[
            pl.BlockSpec((bm, d_in), lambda m: (m, 0)),
            pl.BlockSpec((d_in, d_out), lambda m: (0, 0)),
            pl.BlockSpec((NC, bm, 16), lambda m: (0, m, 0)),
        ],
        out_specs=pl.BlockSpec((bm, d_out), lambda m: (m, 0)),
        out_shape=jax.ShapeDtypeStruct((n, d_out), jnp.float32),
    )(x, W, degp)


def _tc_final(a0, a1, g, degp, b2, pw2, bm):
    n, d_out = g.shape
    half = d_out // 2

    def fin_body(a0_ref, a1_ref, g_ref, d_ref, b_ref, pw_ref, o_ref):
        deg = 1.0 + d_ref[0, :, 0] + d_ref[1, :, 0]
        dinv = lax.rsqrt(deg)[:, None]
        agg = jnp.concatenate([a0_ref[...], a1_ref[...]], axis=1)
        v = dinv * (agg + g_ref[...]) + b_ref[...]
        pw = pw_ref[0, 0]
        o_ref[...] = jnp.where(v >= 0, v, pw * v)

    return pl.pallas_call(
        fin_body,
        grid=(n // bm,),
        in_specs=[
            pl.BlockSpec((bm, half), lambda m: (m, 0)),
            pl.BlockSpec((bm, half), lambda m: (m, 0)),
            pl.BlockSpec((bm, d_out), lambda m: (m, 0)),
            pl.BlockSpec((NC, bm, 16), lambda m: (0, m, 0)),
            pl.BlockSpec((1, d_out), lambda m: (0, 0)),
            pl.BlockSpec(memory_space=pltpu.SMEM),
        ],
        out_specs=pl.BlockSpec((bm, d_out), lambda m: (m, 0)),
        out_shape=jax.ShapeDtypeStruct((n, d_out), jnp.float32),
    )(a0, a1, g, degp, b2, pw2)


def kernel(x, edge_index, W, b, prelu_w):
    n, d_in = x.shape
    d_out = W.shape[1]
    e = edge_index.shape[1]
    half = d_out // 2

    unit = NC * NS * CHUNK
    e_pad = ((e + unit - 1) // unit) * unit
    ch_deg = e_pad // (NC * NS * CHUNK)
    ch_mp = e_pad // (NS * CHUNK)
    rpt = -(-(n + 1) // NS)
    rpt = ((rpt + 7) // 8) * 8
    n_acc = rpt * NS

    src = edge_index[0]
    dst = edge_index[1]
    pad = jnp.full((e_pad - e,), n, dtype=jnp.int32)
    src_p = jnp.concatenate([src, pad])
    dst_p = jnp.concatenate([dst, pad])
    dst_deg = dst_p.reshape(NC * NS, ch_deg, CHUNK)
    src_mp = src_p.reshape(NS, ch_mp, CHUNK)
    dst_mp = dst_p.reshape(NS, ch_mp, CHUNK)

    zeros16 = jnp.zeros((n_acc, 16), jnp.float32)
    ones16 = jnp.ones((CHUNK, 16), jnp.float32)
    zeros_half = jnp.zeros((n_acc, half), jnp.float32)

    degp_full = _sc_degree(dst_deg, zeros16, ones16, n_acc, rpt, ch_deg)
    degp = degp_full[:, :n, :]

    bm = 1000 if n % 1000 == 0 else (500 if n % 500 == 0 else 8)
    g = _tc_matmul(x, W, degp, bm)

    g0 = jnp.pad(g[:, :half], ((0, 8), (0, 0)))
    g1 = jnp.pad(g[:, half:], ((0, 8), (0, 0)))
    agg = _sc_scatter(g0, g1, src_mp, dst_mp, zeros_half, n_acc, rpt, ch_mp,
                      half)

    return _tc_final(agg[0, :n], agg[1, :n], g, degp,
                     b.reshape(1, d_out), prelu_w.reshape(1, 1), bm)


# speedup vs baseline: 16.3911x; 16.3911x over previous
import functools

import jax
import jax.numpy as jnp
from jax import lax
from jax.experimental import pallas as pl
from jax.experimental.pallas import tpu as pltpu
from jax.experimental.pallas import tpu_sc as plsc

NC = 2
NS = 16


def _chunk_size(per_tile, max_c=128, mult=1):
    for c in range(max_c, 0, -1):
        if c % mult == 0 and per_tile % c == 0:
            return c
    return 1


def _sc_degree(dst3, zeros16, ones16, n, rpt, ch_deg):
    mesh = plsc.VectorSubcoreMesh(core_axis_name="c", subcore_axis_name="s",
                                  num_cores=1)

    @functools.partial(
        pl.kernel,
        out_type=jax.ShapeDtypeStruct((1, n, 128), jnp.float32),
        mesh=mesh,
        scratch_types=[
            pltpu.VMEM(dst3.shape[1:], jnp.int32),
            pltpu.VMEM(ones16.shape, jnp.float32),
            pltpu.VMEM_SHARED((n, 128), jnp.float32),
        ],
    )
    def deg_kernel(dst_hbm, zero_hbm, one_hbm, degp_hbm, dstv, onesv, acc):
        s = lax.axis_index("s")
        r0 = s * rpt
        pltpu.sync_copy(zero_hbm.at[pl.ds(r0, rpt)], acc.at[pl.ds(r0, rpt)])
        pltpu.sync_copy(one_hbm, onesv)
        pltpu.sync_copy(dst_hbm.at[s], dstv)
        plsc.subcore_barrier()

        def body(j, carry):
            pltpu.sync_copy(onesv, acc.at[dstv.at[j]], add=True)
            return carry

        lax.fori_loop(0, ch_deg, body, 0)
        plsc.subcore_barrier()
        pltpu.sync_copy(acc.at[pl.ds(r0, rpt)], degp_hbm.at[0, pl.ds(r0, rpt)])

    return deg_kernel(dst3, zeros16, ones16)


def _sc_scatter(g0, g1, src3, dst3, zeros_half, n, rpt, ch_mp, chunk, half):
    mesh = plsc.VectorSubcoreMesh(core_axis_name="c", subcore_axis_name="s")

    per_tile = ch_mp * chunk

    @functools.partial(
        pl.kernel,
        out_type=jax.ShapeDtypeStruct((NC, n, half), jnp.float32),
        mesh=mesh,
        scratch_types=[
            pltpu.VMEM((per_tile,), jnp.int32),
            pltpu.VMEM((ch_mp, chunk), jnp.int32),
            pltpu.VMEM((2, chunk, half), jnp.float32),
            pltpu.VMEM_SHARED((n, half), jnp.float32),
            pltpu.SemaphoreType.DMA,
            pltpu.SemaphoreType.DMA,
        ],
    )
    def mp_kernel(g0_hbm, g1_hbm, src_hbm, dst_hbm, zero_hbm, agg_hbm,
                  srcv, dstv, rowsv, acc, sem0, sem1):
        c = lax.axis_index("c")
        s = lax.axis_index("s")
        r0 = s * rpt
        pltpu.sync_copy(zero_hbm.at[pl.ds(r0, rpt)], acc.at[pl.ds(r0, rpt)])
        pltpu.sync_copy(src_hbm.at[s], srcv)
        pltpu.sync_copy(dst_hbm.at[s], dstv)
        plsc.subcore_barrier()

        def run(g_tab):
            def src_slice(j):
                off = pl.multiple_of(j * chunk, chunk)
                return srcv.at[pl.ds(off, chunk)]

            def step(j, buf, sem):
                pltpu.make_async_copy(g_tab.at[src_slice(j)], rowsv.at[buf],
                                      sem).wait()
                pltpu.sync_copy(rowsv.at[buf], acc.at[dstv.at[j]], add=True)

                @pl.when(j + 2 < ch_mp)
                def _():
                    pltpu.async_copy(g_tab.at[src_slice(j + 2)],
                                     rowsv.at[buf], sem)

            pltpu.async_copy(g_tab.at[src_slice(0)], rowsv.at[0], sem0)
            pltpu.async_copy(g_tab.at[src_slice(1)], rowsv.at[1], sem1)

            def body(j, carry):
                @pl.when(j % 2 == 0)
                def _():
                    step(j, 0, sem0)

                @pl.when(j % 2 == 1)
                def _():
                    step(j, 1, sem1)

                return carry

            lax.fori_loop(0, ch_mp, body, 0)

        @pl.when(c == 0)
        def _():
            run(g0_hbm)

        @pl.when(c == 1)
        def _():
            run(g1_hbm)

        plsc.subcore_barrier()
        pltpu.sync_copy(acc.at[pl.ds(r0, rpt)], agg_hbm.at[c, pl.ds(r0, rpt)])

    return mp_kernel(g0, g1, src3, dst3, zeros_half)


def _tc_matmul(x, W, degp, bm):
    n, d_in = x.shape
    d_out = W.shape[1]
    half = d_out // 2

    def mm_body(x_ref, w_ref, d_ref, g0_ref, g1_ref):
        deg = 1.0 + d_ref[0, :, 0]
        dinv = lax.rsqrt(deg)
        h = jnp.dot(x_ref[...], w_ref[...], preferred_element_type=jnp.float32)
        g = h * dinv[:, None]
        g0_ref[...] = g[:, :half]
        g1_ref[...] = g[:, half:]

    return pl.pallas_call(
        mm_body,
        grid=(n // bm,),
        in_specs=[
            pl.BlockSpec((bm, d_in), lambda m: (m, 0)),
            pl.BlockSpec((d_in, d_out), lambda m: (0, 0)),
            pl.BlockSpec((1, bm, 128), lambda m: (0, m, 0)),
        ],
        out_specs=[
            pl.BlockSpec((bm, half), lambda m: (m, 0)),
            pl.BlockSpec((bm, half), lambda m: (m, 0)),
        ],
        out_shape=[
            jax.ShapeDtypeStruct((n, half), jnp.float32),
            jax.ShapeDtypeStruct((n, half), jnp.float32),
        ],
    )(x, W, degp)


def _tc_final(a0, a1, g0, g1, degp, b2, pw2, bm):
    n, half = g0.shape
    d_out = 2 * half

    def fin_body(a0_ref, a1_ref, g0_ref, g1_ref, d_ref, b_ref, pw_ref, o_ref):
        deg = 1.0 + d_ref[0, :, 0]
        dinv = lax.rsqrt(deg)[:, None]
        agg = jnp.concatenate([a0_ref[...] + g0_ref[...],
                               a1_ref[...] + g1_ref[...]], axis=1)
        v = dinv * agg + b_ref[...]
        pw = pw_ref[0, 0]
        o_ref[...] = jnp.where(v >= 0, v, pw * v)

    return pl.pallas_call(
        fin_body,
        grid=(n // bm,),
        in_specs=[
            pl.BlockSpec((bm, half), lambda m: (m, 0)),
            pl.BlockSpec((bm, half), lambda m: (m, 0)),
            pl.BlockSpec((bm, half), lambda m: (m, 0)),
            pl.BlockSpec((bm, half), lambda m: (m, 0)),
            pl.BlockSpec((1, bm, 128), lambda m: (0, m, 0)),
            pl.BlockSpec((1, d_out), lambda m: (0, 0)),
            pl.BlockSpec(memory_space=pltpu.SMEM),
        ],
        out_specs=pl.BlockSpec((bm, d_out), lambda m: (m, 0)),
        out_shape=jax.ShapeDtypeStruct((n, d_out), jnp.float32),
    )(a0, a1, g0, g1, degp, b2, pw2)


def kernel(x, edge_index, W, b, prelu_w):
    n, d_in = x.shape
    d_out = W.shape[1]
    e = edge_index.shape[1]
    half = d_out // 2

    chunk_mp = _chunk_size(e // NS, max_c=128, mult=8)
    ch_mp = e // (NS * chunk_mp)
    rpt = ((n + NS * 8 - 1) // (NS * 8)) * 8
    n_acc = rpt * NS

    src = edge_index[0]
    dst = edge_index[1]
    src_mp = src.reshape(NS, ch_mp * chunk_mp)
    dst_mp = dst.reshape(NS, ch_mp, chunk_mp)

    zeros16 = jnp.zeros((n_acc, 128), jnp.float32)
    ones16 = jnp.ones((chunk_mp, 128), jnp.float32)
    zeros_half = jnp.zeros((n_acc, half), jnp.float32)

    degp = _sc_degree(dst_mp, zeros16, ones16, n_acc, rpt, ch_mp)[:, :n]

    bm = 1000 if n % 1000 == 0 else (500 if n % 500 == 0 else 8)
    g0, g1 = _tc_matmul(x, W, degp, bm)

    agg = _sc_scatter(g0, g1, src_mp, dst_mp, zeros_half, n_acc, rpt, ch_mp,
                      chunk_mp, half)

    return _tc_final(agg[0, :n], agg[1, :n], g0, g1, degp,
                     b.reshape(1, d_out), prelu_w.reshape(1, 1), bm)


# speedup vs baseline: 17.7098x; 1.0805x over previous
import functools

import jax
import jax.numpy as jnp
from jax import lax
from jax.experimental import pallas as pl
from jax.experimental.pallas import tpu as pltpu
from jax.experimental.pallas import tpu_sc as plsc

NC = 2
NS = 16


def _chunk_size(per_tile, max_c=128, mult=1):
    for c in range(max_c, 0, -1):
        if c % mult == 0 and per_tile % c == 0:
            return c
    return 1


def _sc_degree(dst3, zeros16, ones16, n, rpt, ch_deg):
    mesh = plsc.VectorSubcoreMesh(core_axis_name="c", subcore_axis_name="s")

    @functools.partial(
        pl.kernel,
        out_type=jax.ShapeDtypeStruct((NC, n, 128), jnp.float32),
        mesh=mesh,
        scratch_types=[
            pltpu.VMEM(dst3.shape[1:], jnp.int32),
            pltpu.VMEM(ones16.shape, jnp.float32),
            pltpu.VMEM_SHARED((n, 128), jnp.float32),
        ],
    )
    def deg_kernel(dst_hbm, zero_hbm, one_hbm, degp_hbm, dstv, onesv, acc):
        c = lax.axis_index("c")
        s = lax.axis_index("s")
        r0 = s * rpt
        pltpu.sync_copy(zero_hbm.at[pl.ds(r0, rpt)], acc.at[pl.ds(r0, rpt)])
        pltpu.sync_copy(one_hbm, onesv)
        pltpu.sync_copy(dst_hbm.at[c * NS + s], dstv)
        plsc.subcore_barrier()

        def body(j, carry):
            pltpu.sync_copy(onesv, acc.at[dstv.at[j]], add=True)
            return carry

        lax.fori_loop(0, ch_deg, body, 0)
        plsc.subcore_barrier()
        pltpu.sync_copy(acc.at[pl.ds(r0, rpt)], degp_hbm.at[c, pl.ds(r0, rpt)])

    return deg_kernel(dst3, zeros16, ones16)


def _sc_scatter(g0, g1, src3, dst3, zeros_half, n, rpt, ch_mp, chunk, half):
    mesh = plsc.VectorSubcoreMesh(core_axis_name="c", subcore_axis_name="s")

    per_tile = ch_mp * chunk

    @functools.partial(
        pl.kernel,
        out_type=jax.ShapeDtypeStruct((NC, n, half), jnp.float32),
        mesh=mesh,
        scratch_types=[
            pltpu.VMEM((per_tile,), jnp.int32),
            pltpu.VMEM((ch_mp, chunk), jnp.int32),
            pltpu.VMEM((2, chunk, half), jnp.float32),
            pltpu.VMEM_SHARED((n, half), jnp.float32),
            pltpu.SemaphoreType.DMA,
            pltpu.SemaphoreType.DMA,
        ],
    )
    def mp_kernel(g0_hbm, g1_hbm, src_hbm, dst_hbm, zero_hbm, agg_hbm,
                  srcv, dstv, rowsv, acc, sem0, sem1):
        c = lax.axis_index("c")
        s = lax.axis_index("s")
        r0 = s * rpt
        pltpu.sync_copy(zero_hbm.at[pl.ds(r0, rpt)], acc.at[pl.ds(r0, rpt)])
        pltpu.sync_copy(src_hbm.at[s], srcv)
        pltpu.sync_copy(dst_hbm.at[s], dstv)
        plsc.subcore_barrier()

        def run(g_tab):
            def src_slice(j):
                off = pl.multiple_of(j * chunk, chunk)
                return srcv.at[pl.ds(off, chunk)]

            def step(j, buf, sem):
                pltpu.make_async_copy(g_tab.at[src_slice(j)], rowsv.at[buf],
                                      sem).wait()
                pltpu.sync_copy(rowsv.at[buf], acc.at[dstv.at[j]], add=True)

                @pl.when(j + 2 < ch_mp)
                def _():
                    pltpu.async_copy(g_tab.at[src_slice(j + 2)],
                                     rowsv.at[buf], sem)

            pltpu.async_copy(g_tab.at[src_slice(0)], rowsv.at[0], sem0)
            pltpu.async_copy(g_tab.at[src_slice(1)], rowsv.at[1], sem1)

            def body(j, carry):
                @pl.when(j % 2 == 0)
                def _():
                    step(j, 0, sem0)

                @pl.when(j % 2 == 1)
                def _():
                    step(j, 1, sem1)

                return carry

            lax.fori_loop(0, ch_mp, body, 0)

        @pl.when(c == 0)
        def _():
            run(g0_hbm)

        @pl.when(c == 1)
        def _():
            run(g1_hbm)

        plsc.subcore_barrier()
        pltpu.sync_copy(acc.at[pl.ds(r0, rpt)], agg_hbm.at[c, pl.ds(r0, rpt)])

    return mp_kernel(g0, g1, src3, dst3, zeros_half)


def _tc_matmul(x, W, bm):
    n, d_in = x.shape
    d_out = W.shape[1]
    half = d_out // 2

    def mm_body(x_ref, w_ref, h0_ref, h1_ref):
        h = jnp.dot(x_ref[...], w_ref[...], preferred_element_type=jnp.float32)
        h0_ref[...] = h[:, :half]
        h1_ref[...] = h[:, half:]

    return pl.pallas_call(
        mm_body,
        grid=(n // bm,),
        in_specs=[
            pl.BlockSpec((bm, d_in), lambda m: (m, 0)),
            pl.BlockSpec((d_in, d_out), lambda m: (0, 0)),
        ],
        out_specs=[
            pl.BlockSpec((bm, half), lambda m: (m, 0)),
            pl.BlockSpec((bm, half), lambda m: (m, 0)),
        ],
        out_shape=[
            jax.ShapeDtypeStruct((n, half), jnp.float32),
            jax.ShapeDtypeStruct((n, half), jnp.float32),
        ],
    )(x, W)


def _tc_scale(h0, h1, degp, bm):
    n, half = h0.shape

    def sc_body(h0_ref, h1_ref, d_ref, g0_ref, g1_ref):
        deg = 1.0 + d_ref[0, :, 0] + d_ref[1, :, 0]
        dinv = lax.rsqrt(deg)[:, None]
        g0_ref[...] = h0_ref[...] * dinv
        g1_ref[...] = h1_ref[...] * dinv

    return pl.pallas_call(
        sc_body,
        grid=(n // bm,),
        in_specs=[
            pl.BlockSpec((bm, half), lambda m: (m, 0)),
            pl.BlockSpec((bm, half), lambda m: (m, 0)),
            pl.BlockSpec((NC, bm, 128), lambda m: (0, m, 0)),
        ],
        out_specs=[
            pl.BlockSpec((bm, half), lambda m: (m, 0)),
            pl.BlockSpec((bm, half), lambda m: (m, 0)),
        ],
        out_shape=[
            jax.ShapeDtypeStruct((n, half), jnp.float32),
            jax.ShapeDtypeStruct((n, half), jnp.float32),
        ],
    )(h0, h1, degp)


def _tc_final(a0, a1, g0, g1, degp, b2, pw2, bm):
    n, half = g0.shape
    d_out = 2 * half

    def fin_body(a0_ref, a1_ref, g0_ref, g1_ref, d_ref, b_ref, pw_ref, o_ref):
        deg = 1.0 + d_ref[0, :, 0] + d_ref[1, :, 0]
        dinv = lax.rsqrt(deg)[:, None]
        agg = jnp.concatenate([a0_ref[...] + g0_ref[...],
                               a1_ref[...] + g1_ref[...]], axis=1)
        v = dinv * agg + b_ref[...]
        pw = pw_ref[0, 0]
        o_ref[...] = jnp.where(v >= 0, v, pw * v)

    return pl.pallas_call(
        fin_body,
        grid=(n // bm,),
        in_specs=[
            pl.BlockSpec((bm, half), lambda m: (m, 0)),
            pl.BlockSpec((bm, half), lambda m: (m, 0)),
            pl.BlockSpec((bm, half), lambda m: (m, 0)),
            pl.BlockSpec((bm, half), lambda m: (m, 0)),
            pl.BlockSpec((NC, bm, 128), lambda m: (0, m, 0)),
            pl.BlockSpec((1, d_out), lambda m: (0, 0)),
            pl.BlockSpec(memory_space=pltpu.SMEM),
        ],
        out_specs=pl.BlockSpec((bm, d_out), lambda m: (m, 0)),
        out_shape=jax.ShapeDtypeStruct((n, d_out), jnp.float32),
    )(a0, a1, g0, g1, degp, b2, pw2)


def kernel(x, edge_index, W, b, prelu_w):
    n, d_in = x.shape
    d_out = W.shape[1]
    e = edge_index.shape[1]
    half = d_out // 2

    chunk_mp = _chunk_size(e // NS, max_c=128, mult=8)
    ch_mp = e // (NS * chunk_mp)
    rpt = ((n + NS * 8 - 1) // (NS * 8)) * 8
    n_acc = rpt * NS

    chunk_deg = _chunk_size(e // (NC * NS), max_c=128, mult=8)
    ch_deg = e // (NC * NS * chunk_deg)

    src = edge_index[0]
    dst = edge_index[1]
    src_mp = src.reshape(NS, ch_mp * chunk_mp)
    dst_mp = dst.reshape(NS, ch_mp, chunk_mp)
    dst_deg = dst.reshape(NC * NS, ch_deg, chunk_deg)

    zeros16 = jnp.zeros((n_acc, 128), jnp.float32)
    ones16 = jnp.ones((chunk_deg, 128), jnp.float32)
    zeros_half = jnp.zeros((n_acc, half), jnp.float32)

    degp = _sc_degree(dst_deg, zeros16, ones16, n_acc, rpt, ch_deg)[:, :n]

    bm = 1000 if n % 1000 == 0 else (500 if n % 500 == 0 else 8)
    h0, h1 = _tc_matmul(x, W, bm)
    g0, g1 = _tc_scale(h0, h1, degp, bm)

    agg = _sc_scatter(g0, g1, src_mp, dst_mp, zeros_half, n_acc, rpt, ch_mp,
                      chunk_mp, half)

    return _tc_final(agg[0, :n], agg[1, :n], g0, g1, degp,
                     b.reshape(1, d_out), prelu_w.reshape(1, 1), bm)
